# baseline (speedup 1.0000x reference)
import jax
import jax.numpy as jnp
from jax import lax
from jax.experimental import pallas as pl
from jax.experimental.pallas import tpu as pltpu
from jax.experimental.pallas import tpu_sc as plsc

_NC = 2
_NS = 16
_NW = _NC * _NS
_N = 10000
_E = 320000
_D = 128
_C = 128
_NCHUNK = 80
_EPAD = _NW * _NCHUNK * _C
_NGRP = 2
_GCH = _NCHUNK // _NGRP
_NACC = _N + 8
_NBUF = 2

_RPT = 624
_TAIL_OFF = _RPT * _NS
_TAIL = _N - _TAIL_OFF

_mesh = plsc.VectorSubcoreMesh(core_axis_name="c", subcore_axis_name="s")


def _zero_acc(z128_hbm, acc_sh, sid):
    rb = sid * _RPT
    pltpu.sync_copy(z128_hbm.at[pl.ds(rb, _RPT)], acc_sh.at[pl.ds(rb, _RPT)])

    @pl.when(sid == 0)
    def _():
        pltpu.sync_copy(z128_hbm.at[pl.ds(_TAIL_OFF, _TAIL)],
                        acc_sh.at[pl.ds(_TAIL_OFF, _TAIL)])


def _drain_acc(acc_sh, out, cid, sid):
    rb = sid * _RPT
    pltpu.sync_copy(acc_sh.at[pl.ds(rb, _RPT)], out.at[cid, pl.ds(rb, _RPT)])

    @pl.when(sid == 0)
    def _():
        pltpu.sync_copy(acc_sh.at[pl.ds(_TAIL_OFF, _TAIL)],
                        out.at[cid, pl.ds(_TAIL_OFF, _TAIL)])


def _agg_pipeline(table_hbm, src2_hbm, wid,
                  idx_sa, idx_da, rows, gsems, ssems, acc_sh):
    def g_copy(l, b):
        return pltpu.make_async_copy(table_hbm.at[idx_sa.at[l]],
                                     rows[b], gsems[b])

    def s_copy(j, b):
        return pltpu.make_async_copy(rows[b], acc_sh.at[idx_da.at[j]],
                                     ssems[b])

    for g in range(_NGRP):
        jbase = g * _GCH
        pltpu.sync_copy(src2_hbm.at[wid, pl.ds(jbase, _GCH)], idx_sa)
        g_copy(0, 0).start()

        def body(jj, carry):
            for k in range(_NBUF):
                l = jj * _NBUF + k
                j = jbase + l
                bprv = (k - 1) % _NBUF

                g_copy(l, k).wait()
                s_copy(j, k).start(add=True)

                if k == 0:
                    @pl.when(jj >= 1)
                    def _():
                        s_copy(j - 1, bprv).wait()

                    g_copy(l + 1, bprv).start()
                else:
                    s_copy(j - 1, bprv).wait()

                    @pl.when(jj < _GCH // _NBUF - 1)
                    def _():
                        g_copy(l + 1, bprv).start()
            return carry

        lax.fori_loop(0, _GCH // _NBUF, body, 0)
        s_copy(jbase + _GCH - 1, (_GCH - 1) % _NBUF).wait()


def _deg_pipeline(idx_da, ones_v, ssems, acc_sh):
    def s_copy(j, b):
        return pltpu.make_async_copy(ones_v, acc_sh.at[idx_da.at[j]],
                                     ssems[b])

    def body(jj, carry):
        for k in range(_NBUF):
            j = jj * _NBUF + k

            @pl.when(j >= _NBUF)
            def _():
                s_copy(j - _NBUF, k).wait()

            s_copy(j, k).start(add=True)
        return carry

    lax.fori_loop(0, _NCHUNK // _NBUF, body, 0)
    for k in range(_NBUF):
        s_copy(_NCHUNK - _NBUF + k, k).wait()


def _sc_agg_deg_body(src2_hbm, dst2_hbm, table_hbm, z128_hbm, ones_hbm,
                     acc_out, deg_out,
                     idx_sa, idx_da, rows0, rows1,
                     g0, g1, s0, s1, acc_sh):
    cid = lax.axis_index("c")
    sid = lax.axis_index("s")
    wid = cid * _NS + sid
    rows = [rows0, rows1]
    gsems = [g0, g1]
    ssems = [s0, s1]

    pltpu.sync_copy(dst2_hbm.at[wid], idx_da)
    pltpu.sync_copy(ones_hbm, rows0)

    _zero_acc(z128_hbm, acc_sh, sid)
    plsc.subcore_barrier()
    _deg_pipeline(idx_da, rows0, ssems, acc_sh)
    plsc.subcore_barrier()
    _drain_acc(acc_sh, deg_out, cid, sid)
    plsc.subcore_barrier()

    _zero_acc(z128_hbm, acc_sh, sid)
    plsc.subcore_barrier()
    _agg_pipeline(table_hbm, src2_hbm, wid,
                  idx_sa, idx_da, rows, gsems, ssems, acc_sh)
    plsc.subcore_barrier()
    _drain_acc(acc_sh, acc_out, cid, sid)


def _sc_agg_body(src2_hbm, dst2_hbm, table_hbm, z128_hbm, acc_out,
                 idx_sa, idx_da, rows0, rows1,
                 g0, g1, s0, s1, acc_sh):
    cid = lax.axis_index("c")
    sid = lax.axis_index("s")
    wid = cid * _NS + sid
    rows = [rows0, rows1]
    gsems = [g0, g1]
    ssems = [s0, s1]

    pltpu.sync_copy(dst2_hbm.at[wid], idx_da)

    _zero_acc(z128_hbm, acc_sh, sid)
    plsc.subcore_barrier()
    _agg_pipeline(table_hbm, src2_hbm, wid,
                  idx_sa, idx_da, rows, gsems, ssems, acc_sh)
    plsc.subcore_barrier()
    _drain_acc(acc_sh, acc_out, cid, sid)


def _sc_scratch():
    return [
        pltpu.VMEM((_GCH, _C), jnp.int32),
        pltpu.VMEM((_NCHUNK, _C), jnp.int32),
        pltpu.VMEM((_C, _D), jnp.float32),
        pltpu.VMEM((_C, _D), jnp.float32),
        pltpu.SemaphoreType.DMA,
        pltpu.SemaphoreType.DMA,
        pltpu.SemaphoreType.DMA,
        pltpu.SemaphoreType.DMA,
        pltpu.VMEM_SHARED((_NACC, _D), jnp.float32),
    ]


_sc_agg_deg = pl.kernel(
    _sc_agg_deg_body,
    mesh=_mesh,
    out_type=[
        jax.ShapeDtypeStruct((_NC, _N, _D), jnp.float32),
        jax.ShapeDtypeStruct((_NC, _N, _D), jnp.float32),
    ],
    scratch_types=_sc_scratch(),
)

_sc_agg = pl.kernel(
    _sc_agg_body,
    mesh=_mesh,
    out_type=[
        jax.ShapeDtypeStruct((_NC, _N, _D), jnp.float32),
    ],
    scratch_types=_sc_scratch(),
)

_R = 1000


def _tc_mid_body(x_ref, a0_ref, a1_ref, d0_ref, d1_ref,
                 ws1_ref, wn1_ref, b1_ref, ws2_ref, wn2_ref, b2_ref,
                 p2_ref, hs2_ref):
    deg = jnp.maximum(d0_ref[:, 0:1] + d1_ref[:, 0:1], 1.0)
    hn1 = (a0_ref[...] + a1_ref[...]) / deg
    h1 = (jnp.dot(x_ref[...], ws1_ref[...], preferred_element_type=jnp.float32)
          + jnp.dot(hn1, wn1_ref[...], preferred_element_type=jnp.float32)
          + b1_ref[...])
    h1 = jnp.maximum(h1, 0.0)
    p2_ref[...] = jnp.dot(h1, wn2_ref[...], preferred_element_type=jnp.float32)
    hs2_ref[...] = (jnp.dot(h1, ws2_ref[...], preferred_element_type=jnp.float32)
                    + b2_ref[...])


def _tc_fin_body(hs2_ref, a0_ref, a1_ref, d0_ref, d1_ref, o_ref):
    deg = jnp.maximum(d0_ref[:, 0:1] + d1_ref[:, 0:1], 1.0)
    o_ref[...] = hs2_ref[...] + (a0_ref[...] + a1_ref[...]) / deg


def _row_spec(w):
    return pl.BlockSpec((_R, w), lambda i: (i, 0))


def _full_spec(h, w):
    return pl.BlockSpec((h, w), lambda i: (0, 0))


_tc_mid = pl.pallas_call(
    _tc_mid_body,
    grid=(_N // _R,),
    in_specs=[
        _row_spec(128), _row_spec(128), _row_spec(128),
        _row_spec(128), _row_spec(128),
        _full_spec(128, 256), _full_spec(128, 256), _full_spec(1, 256),
        _full_spec(256, 128), _full_spec(256, 128), _full_spec(1, 128),
    ],
    out_specs=[_row_spec(128), _row_spec(128)],
    out_shape=[
        jax.ShapeDtypeStruct((_N, 128), jnp.float32),
        jax.ShapeDtypeStruct((_N, 128), jnp.float32),
    ],
)

_tc_fin = pl.pallas_call(
    _tc_fin_body,
    grid=(_N // _R,),
    in_specs=[
        _row_spec(128), _row_spec(128), _row_spec(128),
        _row_spec(128), _row_spec(128),
    ],
    out_specs=_row_spec(128),
    out_shape=jax.ShapeDtypeStruct((_N, 128), jnp.float32),
)


def kernel(in_feat, edge_index, W_self1, W_neigh1, b1, W_self2, W_neigh2, b2):
    src = edge_index[0].astype(jnp.int32)
    dst = edge_index[1].astype(jnp.int32)
    npad = _EPAD - _E
    src = jnp.concatenate([src, jnp.zeros((npad,), jnp.int32)])
    dst = jnp.concatenate([dst, jnp.full((npad,), _N, jnp.int32)])
    src = src.reshape(_NW, _NCHUNK, _C)
    dst = dst.reshape(_NW, _NCHUNK, _C)
    z128 = jnp.zeros((_N, _D), jnp.float32)
    ones = jnp.ones((_C, _D), jnp.float32)

    acc1, degp = _sc_agg_deg(src, dst, in_feat, z128, ones)
    p2, hs2 = _tc_mid(in_feat, acc1[0], acc1[1], degp[0], degp[1],
                      W_self1, W_neigh1, b1.reshape(1, -1),
                      W_self2, W_neigh2, b2.reshape(1, -1))
    (acc2,) = _sc_agg(src, dst, p2, z128)
    return _tc_fin(hs2, acc2[0], acc2[1], degp[0], degp[1])


# speedup vs baseline: 1.1287x; 1.1287x over previous
import jax
import jax.numpy as jnp
from jax import lax
from jax.experimental import pallas as pl
from jax.experimental.pallas import tpu as pltpu
from jax.experimental.pallas import tpu_sc as plsc

_NC = 2
_NS = 16
_NW = _NC * _NS
_N = 10000
_E = 320000
_D = 128
_C = 128
_NCHUNK = 80
_EPAD = _NW * _NCHUNK * _C
_NGRP = 2
_GCH = _NCHUNK // _NGRP
_NACC = _N + 16
_PER_TILE = _E // _NW
_PADT = _NCHUNK * _C - _PER_TILE
_NBUF = 2

_RPT = 624
_TAIL_OFF = _RPT * _NS
_TAIL = _N - _TAIL_OFF

_mesh = plsc.VectorSubcoreMesh(core_axis_name="c", subcore_axis_name="s")


def _zero_acc(z128_hbm, acc_sh, sid):
    rb = sid * _RPT
    pltpu.sync_copy(z128_hbm.at[pl.ds(rb, _RPT)], acc_sh.at[pl.ds(rb, _RPT)])

    @pl.when(sid == 0)
    def _():
        pltpu.sync_copy(z128_hbm.at[pl.ds(_TAIL_OFF, _TAIL)],
                        acc_sh.at[pl.ds(_TAIL_OFF, _TAIL)])


def _drain_acc(acc_sh, out, cid, sid):
    rb = sid * _RPT
    pltpu.sync_copy(acc_sh.at[pl.ds(rb, _RPT)], out.at[cid, pl.ds(rb, _RPT)])

    @pl.when(sid == 0)
    def _():
        pltpu.sync_copy(acc_sh.at[pl.ds(_TAIL_OFF, _TAIL)],
                        out.at[cid, pl.ds(_TAIL_OFF, _TAIL)])


def _agg_pipeline(table_hbm, src2_hbm, wid,
                  idx_sa, idx_da, rows, gsems, ssems, acc_sh):
    def g_copy(l, b):
        return pltpu.make_async_copy(table_hbm.at[idx_sa.at[l]],
                                     rows[b], gsems[b])

    def s_copy(j, b):
        return pltpu.make_async_copy(rows[b], acc_sh.at[idx_da.at[j]],
                                     ssems[b])

    for g in range(_NGRP):
        jbase = g * _GCH
        pltpu.sync_copy(src2_hbm.at[wid, pl.ds(jbase, _GCH)], idx_sa)
        g_copy(0, 0).start()

        def body(jj, carry):
            for k in range(_NBUF):
                l = jj * _NBUF + k
                j = jbase + l
                bprv = (k - 1) % _NBUF

                g_copy(l, k).wait()
                s_copy(j, k).start(add=True)

                if k == 0:
                    @pl.when(jj >= 1)
                    def _():
                        s_copy(j - 1, bprv).wait()

                    g_copy(l + 1, bprv).start()
                else:
                    s_copy(j - 1, bprv).wait()

                    @pl.when(jj < _GCH // _NBUF - 1)
                    def _():
                        g_copy(l + 1, bprv).start()
            return carry

        lax.fori_loop(0, _GCH // _NBUF, body, 0)
        s_copy(jbase + _GCH - 1, (_GCH - 1) % _NBUF).wait()


def _deg_pipeline(idx_da, ones_v, ssems, acc_sh):
    def s_copy(j, b):
        return pltpu.make_async_copy(ones_v, acc_sh.at[idx_da.at[j]],
                                     ssems[b])

    def body(jj, carry):
        for k in range(_NBUF):
            j = jj * _NBUF + k

            @pl.when(j >= _NBUF)
            def _():
                s_copy(j - _NBUF, k).wait()

            s_copy(j, k).start(add=True)
        return carry

    lax.fori_loop(0, _NCHUNK // _NBUF, body, 0)
    for k in range(_NBUF):
        s_copy(_NCHUNK - _NBUF + k, k).wait()


def _sc_agg_deg_body(src2_hbm, dst2_hbm, table_hbm, z128_hbm, ones_hbm,
                     acc_out, deg_out,
                     idx_sa, idx_da, rows0, rows1,
                     g0, g1, s0, s1, acc_sh):
    cid = lax.axis_index("c")
    sid = lax.axis_index("s")
    wid = cid * _NS + sid
    rows = [rows0, rows1]
    gsems = [g0, g1]
    ssems = [s0, s1]

    pltpu.sync_copy(dst2_hbm.at[wid], idx_da)
    pltpu.sync_copy(ones_hbm, rows0)

    _zero_acc(z128_hbm, acc_sh, sid)
    plsc.subcore_barrier()
    _deg_pipeline(idx_da, rows0, ssems, acc_sh)
    plsc.subcore_barrier()
    _drain_acc(acc_sh, deg_out, cid, sid)
    plsc.subcore_barrier()

    _zero_acc(z128_hbm, acc_sh, sid)
    plsc.subcore_barrier()
    _agg_pipeline(table_hbm, src2_hbm, wid,
                  idx_sa, idx_da, rows, gsems, ssems, acc_sh)
    plsc.subcore_barrier()
    _drain_acc(acc_sh, acc_out, cid, sid)


def _sc_agg_body(src2_hbm, dst2_hbm, table_hbm, z128_hbm, acc_out,
                 idx_sa, idx_da, rows0, rows1,
                 g0, g1, s0, s1, acc_sh):
    cid = lax.axis_index("c")
    sid = lax.axis_index("s")
    wid = cid * _NS + sid
    rows = [rows0, rows1]
    gsems = [g0, g1]
    ssems = [s0, s1]

    pltpu.sync_copy(dst2_hbm.at[wid], idx_da)

    _zero_acc(z128_hbm, acc_sh, sid)
    plsc.subcore_barrier()
    _agg_pipeline(table_hbm, src2_hbm, wid,
                  idx_sa, idx_da, rows, gsems, ssems, acc_sh)
    plsc.subcore_barrier()
    _drain_acc(acc_sh, acc_out, cid, sid)


def _sc_scratch():
    return [
        pltpu.VMEM((_GCH, _C), jnp.int32),
        pltpu.VMEM((_NCHUNK, _C), jnp.int32),
        pltpu.VMEM((_C, _D), jnp.float32),
        pltpu.VMEM((_C, _D), jnp.float32),
        pltpu.SemaphoreType.DMA,
        pltpu.SemaphoreType.DMA,
        pltpu.SemaphoreType.DMA,
        pltpu.SemaphoreType.DMA,
        pltpu.VMEM_SHARED((_NACC, _D), jnp.float32),
    ]


_sc_agg_deg = pl.kernel(
    _sc_agg_deg_body,
    mesh=_mesh,
    out_type=[
        jax.ShapeDtypeStruct((_NC, _N, _D), jnp.float32),
        jax.ShapeDtypeStruct((_NC, _N, _D), jnp.float32),
    ],
    scratch_types=_sc_scratch(),
)

_sc_agg = pl.kernel(
    _sc_agg_body,
    mesh=_mesh,
    out_type=[
        jax.ShapeDtypeStruct((_NC, _N, _D), jnp.float32),
    ],
    scratch_types=_sc_scratch(),
)

_R = 1000


def _tc_mid_body(x_ref, a0_ref, a1_ref, d0_ref, d1_ref,
                 ws1_ref, wn1_ref, b1_ref, ws2_ref, wn2_ref, b2_ref,
                 p2_ref, hs2_ref):
    deg = jnp.maximum(d0_ref[:, 0:1] + d1_ref[:, 0:1], 1.0)
    hn1 = (a0_ref[...] + a1_ref[...]) / deg
    h1 = (jnp.dot(x_ref[...], ws1_ref[...], preferred_element_type=jnp.float32)
          + jnp.dot(hn1, wn1_ref[...], preferred_element_type=jnp.float32)
          + b1_ref[...])
    h1 = jnp.maximum(h1, 0.0)
    p2_ref[...] = jnp.dot(h1, wn2_ref[...], preferred_element_type=jnp.float32)
    hs2_ref[...] = (jnp.dot(h1, ws2_ref[...], preferred_element_type=jnp.float32)
                    + b2_ref[...])


def _tc_fin_body(hs2_ref, a0_ref, a1_ref, d0_ref, d1_ref, o_ref):
    deg = jnp.maximum(d0_ref[:, 0:1] + d1_ref[:, 0:1], 1.0)
    o_ref[...] = hs2_ref[...] + (a0_ref[...] + a1_ref[...]) / deg


def _row_spec(w):
    return pl.BlockSpec((_R, w), lambda i: (i, 0))


def _full_spec(h, w):
    return pl.BlockSpec((h, w), lambda i: (0, 0))


_tc_mid = pl.pallas_call(
    _tc_mid_body,
    grid=(_N // _R,),
    in_specs=[
        _row_spec(128), _row_spec(128), _row_spec(128),
        _row_spec(128), _row_spec(128),
        _full_spec(128, 256), _full_spec(128, 256), _full_spec(1, 256),
        _full_spec(256, 128), _full_spec(256, 128), _full_spec(1, 128),
    ],
    out_specs=[_row_spec(128), _row_spec(128)],
    out_shape=[
        jax.ShapeDtypeStruct((_N, 128), jnp.float32),
        jax.ShapeDtypeStruct((_N, 128), jnp.float32),
    ],
)

_tc_fin = pl.pallas_call(
    _tc_fin_body,
    grid=(_N // _R,),
    in_specs=[
        _row_spec(128), _row_spec(128), _row_spec(128),
        _row_spec(128), _row_spec(128),
    ],
    out_specs=_row_spec(128),
    out_shape=jax.ShapeDtypeStruct((_N, 128), jnp.float32),
)


def kernel(in_feat, edge_index, W_self1, W_neigh1, b1, W_self2, W_neigh2, b2):
    src = edge_index[0].astype(jnp.int32).reshape(_NW, _PER_TILE)
    dst = edge_index[1].astype(jnp.int32).reshape(_NW, _PER_TILE)
    pad_src = jnp.zeros((_NW, _PADT), jnp.int32)
    pad_dst = jnp.broadcast_to(_N + (jnp.arange(_PADT, dtype=jnp.int32) % 16),
                               (_NW, _PADT))
    src = jnp.concatenate([src, pad_src], axis=1).reshape(_NW, _NCHUNK, _C)
    dst = jnp.concatenate([dst, pad_dst], axis=1).reshape(_NW, _NCHUNK, _C)
    z128 = jnp.zeros((_N, _D), jnp.float32)
    ones = jnp.ones((_C, _D), jnp.float32)

    acc1, degp = _sc_agg_deg(src, dst, in_feat, z128, ones)
    p2, hs2 = _tc_mid(in_feat, acc1[0], acc1[1], degp[0], degp[1],
                      W_self1, W_neigh1, b1.reshape(1, -1),
                      W_self2, W_neigh2, b2.reshape(1, -1))
    (acc2,) = _sc_agg(src, dst, p2, z128)
    return _tc_fin(hs2, acc2[0], acc2[1], degp[0], degp[1])


# speedup vs baseline: 2.1342x; 1.8908x over previous
import jax
import jax.numpy as jnp
from jax import lax
from jax.experimental import pallas as pl
from jax.experimental.pallas import tpu as pltpu
from jax.experimental.pallas import tpu_sc as plsc

_NC = 2
_NS = 16
_NW = _NC * _NS
_N = 10000
_E = 320000
_D = 128
_C = 80
_PER_TILE = _E // _NW
_NCHUNK = _PER_TILE // _C
_NACC = _N
_NBUF = 2

_RPT = 624
_TAIL_OFF = _RPT * _NS
_TAIL = _N - _TAIL_OFF

_mesh = plsc.VectorSubcoreMesh(core_axis_name="c", subcore_axis_name="s")


def _zero_acc(z128_hbm, acc_sh, sid):
    rb = sid * _RPT
    pltpu.sync_copy(z128_hbm.at[pl.ds(rb, _RPT)], acc_sh.at[pl.ds(rb, _RPT)])

    @pl.when(sid == 0)
    def _():
        pltpu.sync_copy(z128_hbm.at[pl.ds(_TAIL_OFF, _TAIL)],
                        acc_sh.at[pl.ds(_TAIL_OFF, _TAIL)])


def _drain_acc(acc_sh, out, cid, sid):
    rb = sid * _RPT
    pltpu.sync_copy(acc_sh.at[pl.ds(rb, _RPT)], out.at[cid, pl.ds(rb, _RPT)])

    @pl.when(sid == 0)
    def _():
        pltpu.sync_copy(acc_sh.at[pl.ds(_TAIL_OFF, _TAIL)],
                        out.at[cid, pl.ds(_TAIL_OFF, _TAIL)])


def _agg_pipeline(src_hbm, dst_hbm, table_hbm, base,
                  idx_s, idx_d, rows, gsems, ssems, acc_sh):
    def g_copy(b):
        return pltpu.make_async_copy(table_hbm.at[idx_s[b]],
                                     rows[b], gsems[b])

    def s_copy(b):
        return pltpu.make_async_copy(rows[b], acc_sh.at[idx_d[b]],
                                     ssems[b])

    def load_idx(j, b):
        off = base + j * _C
        pltpu.sync_copy(src_hbm.at[pl.ds(off, _C)], idx_s[b])
        pltpu.sync_copy(dst_hbm.at[pl.ds(off, _C)], idx_d[b])

    load_idx(0, 0)
    g_copy(0).start()

    def body(jj, carry):
        for k in range(_NBUF):
            j = jj * _NBUF + k
            b = k
            bprv = (k - 1) % _NBUF

            @pl.when(j < _NCHUNK)
            def _():
                @pl.when(j >= 1)
                def _():
                    s_copy(bprv).wait()

                @pl.when(j + 1 < _NCHUNK)
                def _():
                    load_idx(j + 1, bprv)

                g_copy(b).wait()
                s_copy(b).start(add=True)

                @pl.when(j + 1 < _NCHUNK)
                def _():
                    g_copy(bprv).start()
        return carry

    lax.fori_loop(0, (_NCHUNK + _NBUF - 1) // _NBUF, body, 0)
    s_copy((_NCHUNK - 1) % _NBUF).wait()


def _deg_pipeline(dst_hbm, base, idx_d, ones_v, ssems, acc_sh):
    def s_copy(b):
        return pltpu.make_async_copy(ones_v, acc_sh.at[idx_d[b]],
                                     ssems[b])

    def load_idx(j, b):
        pltpu.sync_copy(dst_hbm.at[pl.ds(base + j * _C, _C)], idx_d[b])

    load_idx(0, 0)
    s_copy(0).start(add=True)

    def body(jj, carry):
        for k in range(_NBUF):
            j = jj * _NBUF + k
            bprv = (k - 1) % _NBUF

            @pl.when(j + 1 < _NCHUNK)
            def _():
                @pl.when(j >= 1)
                def _():
                    s_copy(bprv).wait()

                load_idx(j + 1, bprv)
                s_copy(bprv).start(add=True)
        return carry

    lax.fori_loop(0, (_NCHUNK + _NBUF - 1) // _NBUF, body, 0)
    s_copy((_NCHUNK - 2) % _NBUF).wait()
    s_copy((_NCHUNK - 1) % _NBUF).wait()


def _sc_agg_deg_body(src_hbm, dst_hbm, table_hbm, z128_hbm, ones_hbm,
                     acc_out, deg_out,
                     is0, is1, id0, id1, rows0, rows1,
                     g0, g1, s0, s1, acc_sh):
    cid = lax.axis_index("c")
    sid = lax.axis_index("s")
    wid = cid * _NS + sid
    base = wid * _PER_TILE
    idx_s = [is0, is1]
    idx_d = [id0, id1]
    rows = [rows0, rows1]
    gsems = [g0, g1]
    ssems = [s0, s1]

    pltpu.sync_copy(ones_hbm, rows0)

    _zero_acc(z128_hbm, acc_sh, sid)
    plsc.subcore_barrier()
    _deg_pipeline(dst_hbm, base, idx_d, rows0, ssems, acc_sh)
    plsc.subcore_barrier()
    _drain_acc(acc_sh, deg_out, cid, sid)
    plsc.subcore_barrier()

    _zero_acc(z128_hbm, acc_sh, sid)
    plsc.subcore_barrier()
    _agg_pipeline(src_hbm, dst_hbm, table_hbm, base,
                  idx_s, idx_d, rows, gsems, ssems, acc_sh)
    plsc.subcore_barrier()
    _drain_acc(acc_sh, acc_out, cid, sid)


def _sc_agg_body(src_hbm, dst_hbm, table_hbm, z128_hbm, acc_out,
                 is0, is1, id0, id1, rows0, rows1,
                 g0, g1, s0, s1, acc_sh):
    cid = lax.axis_index("c")
    sid = lax.axis_index("s")
    wid = cid * _NS + sid
    base = wid * _PER_TILE
    idx_s = [is0, is1]
    idx_d = [id0, id1]
    rows = [rows0, rows1]
    gsems = [g0, g1]
    ssems = [s0, s1]

    _zero_acc(z128_hbm, acc_sh, sid)
    plsc.subcore_barrier()
    _agg_pipeline(src_hbm, dst_hbm, table_hbm, base,
                  idx_s, idx_d, rows, gsems, ssems, acc_sh)
    plsc.subcore_barrier()
    _drain_acc(acc_sh, acc_out, cid, sid)


def _sc_scratch():
    return [
        pltpu.VMEM((_C,), jnp.int32),
        pltpu.VMEM((_C,), jnp.int32),
        pltpu.VMEM((_C,), jnp.int32),
        pltpu.VMEM((_C,), jnp.int32),
        pltpu.VMEM((_C, _D), jnp.float32),
        pltpu.VMEM((_C, _D), jnp.float32),
        pltpu.SemaphoreType.DMA,
        pltpu.SemaphoreType.DMA,
        pltpu.SemaphoreType.DMA,
        pltpu.SemaphoreType.DMA,
        pltpu.VMEM_SHARED((_NACC, _D), jnp.float32),
    ]


_sc_agg_deg = pl.kernel(
    _sc_agg_deg_body,
    mesh=_mesh,
    out_type=[
        jax.ShapeDtypeStruct((_NC, _N, _D), jnp.float32),
        jax.ShapeDtypeStruct((_NC, _N, _D), jnp.float32),
    ],
    scratch_types=_sc_scratch(),
)

_sc_agg = pl.kernel(
    _sc_agg_body,
    mesh=_mesh,
    out_type=[
        jax.ShapeDtypeStruct((_NC, _N, _D), jnp.float32),
    ],
    scratch_types=_sc_scratch(),
)

_R = 1000


def _tc_mid_body(x_ref, a0_ref, a1_ref, d0_ref, d1_ref,
                 ws1_ref, wn1_ref, b1_ref, ws2_ref, wn2_ref, b2_ref,
                 p2_ref, hs2_ref):
    deg = jnp.maximum(d0_ref[:, 0:1] + d1_ref[:, 0:1], 1.0)
    hn1 = (a0_ref[...] + a1_ref[...]) / deg
    h1 = (jnp.dot(x_ref[...], ws1_ref[...], preferred_element_type=jnp.float32)
          + jnp.dot(hn1, wn1_ref[...], preferred_element_type=jnp.float32)
          + b1_ref[...])
    h1 = jnp.maximum(h1, 0.0)
    p2_ref[...] = jnp.dot(h1, wn2_ref[...], preferred_element_type=jnp.float32)
    hs2_ref[...] = (jnp.dot(h1, ws2_ref[...], preferred_element_type=jnp.float32)
                    + b2_ref[...])


def _tc_fin_body(hs2_ref, a0_ref, a1_ref, d0_ref, d1_ref, o_ref):
    deg = jnp.maximum(d0_ref[:, 0:1] + d1_ref[:, 0:1], 1.0)
    o_ref[...] = hs2_ref[...] + (a0_ref[...] + a1_ref[...]) / deg


def _row_spec(w):
    return pl.BlockSpec((_R, w), lambda i: (i, 0))


def _full_spec(h, w):
    return pl.BlockSpec((h, w), lambda i: (0, 0))


_tc_mid = pl.pallas_call(
    _tc_mid_body,
    grid=(_N // _R,),
    in_specs=[
        _row_spec(128), _row_spec(128), _row_spec(128),
        _row_spec(128), _row_spec(128),
        _full_spec(128, 256), _full_spec(128, 256), _full_spec(1, 256),
        _full_spec(256, 128), _full_spec(256, 128), _full_spec(1, 128),
    ],
    out_specs=[_row_spec(128), _row_spec(128)],
    out_shape=[
        jax.ShapeDtypeStruct((_N, 128), jnp.float32),
        jax.ShapeDtypeStruct((_N, 128), jnp.float32),
    ],
)

_tc_fin = pl.pallas_call(
    _tc_fin_body,
    grid=(_N // _R,),
    in_specs=[
        _row_spec(128), _row_spec(128), _row_spec(128),
        _row_spec(128), _row_spec(128),
    ],
    out_specs=_row_spec(128),
    out_shape=jax.ShapeDtypeStruct((_N, 128), jnp.float32),
)


def kernel(in_feat, edge_index, W_self1, W_neigh1, b1, W_self2, W_neigh2, b2):
    src = edge_index[0].astype(jnp.int32)
    dst = edge_index[1].astype(jnp.int32)
    z128 = jnp.zeros((_N, _D), jnp.float32)
    ones = jnp.ones((_C, _D), jnp.float32)

    acc1, degp = _sc_agg_deg(src, dst, in_feat, z128, ones)
    p2, hs2 = _tc_mid(in_feat, acc1[0], acc1[1], degp[0], degp[1],
                      W_self1, W_neigh1, b1.reshape(1, -1),
                      W_self2, W_neigh2, b2.reshape(1, -1))
    (acc2,) = _sc_agg(src, dst, p2, z128)
    return _tc_fin(hs2, acc2[0], acc2[1], degp[0], degp[1])
